# baseline (speedup 1.0000x reference)
import functools

import jax
import jax.numpy as jnp
from jax import lax
from jax.experimental import pallas as pl
from jax.experimental.pallas import tpu as pltpu
from jax.experimental.pallas import tpu_sc as plsc

S = 2048
H = 128
VOCAB = 100000
TILE_V = 512
NV = (VOCAB + TILE_V - 1) // TILE_V

NC = 2
NS = 16
NW = NC * NS
BPW = S // NW


def _vocab_body(x_ref, g_ref, b_ref, w_ref, logits_ref, xn_ref, dec_ref,
                xn_s, rmax_s, rarg_s):
    v = pl.program_id(0)

    @pl.when(v == 0)
    def _init():
        x = x_ref[...]
        mu = jnp.mean(x, axis=1, keepdims=True)
        var = jnp.mean((x - mu) ** 2, axis=1, keepdims=True)
        xn = (x - mu) / jnp.sqrt(var + 1e-5) * g_ref[...] + b_ref[...]
        xn_s[...] = xn
        xn_ref[...] = xn
        rmax_s[...] = jnp.full((S, 1), -jnp.inf, jnp.float32)
        rarg_s[...] = jnp.zeros((S, 1), jnp.int32)

    xn = xn_s[...]
    logits = lax.dot_general(xn, w_ref[...], (((1,), (1,)), ((), ())),
                             preferred_element_type=jnp.float32)
    logits_ref[...] = logits

    ids = lax.broadcasted_iota(jnp.int32, (S, TILE_V), 1)
    limit = VOCAB - v * TILE_V
    masked = jnp.where(ids < limit, logits, -jnp.inf)
    m = jnp.max(masked, axis=1, keepdims=True)
    amin = jnp.min(jnp.where(masked == m, ids, TILE_V), axis=1, keepdims=True)
    better = m > rmax_s[...]
    rarg_s[...] = jnp.where(better, amin + v * TILE_V, rarg_s[...])
    rmax_s[...] = jnp.maximum(m, rmax_s[...])

    @pl.when(v == NV - 1)
    def _fin():
        dec_ref[...] = rarg_s[...]


_VOCAB_KWARGS = dict(
    grid=(NV,),
    in_specs=[
        pl.BlockSpec((S, H), lambda v: (0, 0)),
        pl.BlockSpec((1, H), lambda v: (0, 0)),
        pl.BlockSpec((1, H), lambda v: (0, 0)),
        pl.BlockSpec((TILE_V, H), lambda v: (v, 0)),
    ],
    out_specs=[
        pl.BlockSpec((S, TILE_V), lambda v: (0, v)),
        pl.BlockSpec((S, H), lambda v: (0, 0)),
        pl.BlockSpec((S, 1), lambda v: (0, 0)),
    ],
    out_shape=[
        jax.ShapeDtypeStruct((S, VOCAB), jnp.float32),
        jax.ShapeDtypeStruct((S, H), jnp.float32),
        jax.ShapeDtypeStruct((S, 1), jnp.int32),
    ],
    scratch_shapes=[
        pltpu.VMEM((S, H), jnp.float32),
        pltpu.VMEM((S, 1), jnp.float32),
        pltpu.VMEM((S, 1), jnp.int32),
    ],
    compiler_params=pltpu.CompilerParams(
        dimension_semantics=("arbitrary",)),
)
_vocab_call = pl.pallas_call(_vocab_body, **_VOCAB_KWARGS)


@functools.cache
def _sc_gather_call():
    mesh = plsc.VectorSubcoreMesh(core_axis_name="c", subcore_axis_name="s")

    @functools.partial(
        pl.kernel,
        mesh=mesh,
        out_type=jax.ShapeDtypeStruct((S, H), jnp.float32),
        scratch_types=[
            pltpu.VMEM((BPW,), jnp.int32),
            pltpu.VMEM((BPW, H), jnp.float32),
            pltpu.SemaphoreType.DMA,
        ],
    )
    def _sc_gather(table_hbm, idx_hbm, out_hbm, idx_v, rows_v, sem):
        wid = lax.axis_index("s") * NC + lax.axis_index("c")
        base = wid * BPW
        pltpu.sync_copy(idx_hbm.at[pl.ds(base, BPW)], idx_v)
        pltpu.async_copy(table_hbm.at[idx_v], rows_v, sem).wait()
        pltpu.sync_copy(rows_v, out_hbm.at[pl.ds(base, BPW)])

    return _sc_gather


def _lane_argmax(logits, n):
    ids = lax.broadcasted_iota(jnp.int32, (S, n), 1)
    m = jnp.max(logits, axis=1, keepdims=True)
    return jnp.min(jnp.where(logits == m, ids, n), axis=1, keepdims=True)


def _head_body(xn_ref, emb_ref, w1a_ref, w1b_ref, b1_ref, w2_ref, b2_ref,
               w2e_ref, b2e_ref, w2o_ref, b2o_ref, wea_ref, bea_ref,
               raw_ref, ea_ref, evarg_ref, odarg_ref, eaarg_ref):
    xn = xn_ref[...]
    emb = emb_ref[...]
    h = jnp.maximum(
        jnp.dot(xn, w1a_ref[...], preferred_element_type=jnp.float32)
        + jnp.dot(emb, w1b_ref[...], preferred_element_type=jnp.float32)
        + b1_ref[...], 0.0)
    raw_ref[...] = (jnp.dot(h, w2_ref[...], preferred_element_type=jnp.float32)
                    + b2_ref[...])
    ea = (jnp.dot(xn, wea_ref[...], preferred_element_type=jnp.float32)
          + bea_ref[...])
    ea_ref[...] = ea
    even = (jnp.dot(h, w2e_ref[...], preferred_element_type=jnp.float32)
            + b2e_ref[...])
    odd = (jnp.dot(h, w2o_ref[...], preferred_element_type=jnp.float32)
           + b2o_ref[...])
    evarg_ref[...] = _lane_argmax(even, H)
    odarg_ref[...] = _lane_argmax(odd, H)
    eaarg_ref[...] = _lane_argmax(ea, 16)


_HEAD_KWARGS = dict(
    out_shape=[
        jax.ShapeDtypeStruct((S, 2 * H), jnp.float32),
        jax.ShapeDtypeStruct((S, 16), jnp.float32),
        jax.ShapeDtypeStruct((S, 1), jnp.int32),
        jax.ShapeDtypeStruct((S, 1), jnp.int32),
        jax.ShapeDtypeStruct((S, 1), jnp.int32),
    ],
)
_head_call = pl.pallas_call(_head_body, **_HEAD_KWARGS)


def kernel(llama_output, ln_gamma, ln_beta, w_node_dec, node_emb,
           w1, b1, w2, b2, w_ea, b_ea):
    x = llama_output.reshape(S, H)
    gamma = ln_gamma.reshape(1, H)
    beta = ln_beta.reshape(1, H)

    node_logits, xn, decoded = _vocab_call(x, gamma, beta, w_node_dec)

    emb = _sc_gather_call()(node_emb, decoded.reshape(S))

    raw, ea, evarg, odarg, eaarg = _head_call(
        xn, emb,
        w1[:, :H].T, w1[:, H:].T, b1.reshape(1, H),
        w2.T, b2.reshape(1, 2 * H),
        w2[0::2].T, b2[0::2].reshape(1, H),
        w2[1::2].T, b2[1::2].reshape(1, H),
        w_ea.T, b_ea.reshape(1, 16),
    )

    return (
        node_logits.reshape(1, S, VOCAB),
        raw.reshape(1, S, H, 2),
        ea.reshape(1, S, 16),
        decoded.reshape(1, S),
        jnp.concatenate([evarg, odarg], axis=1).reshape(1, S, 2),
        eaarg.reshape(1, S),
    )
